# baseline (speedup 1.0000x reference)
import functools

import jax
import jax.numpy as jnp
from jax import lax
from jax.experimental import pallas as pl
from jax.experimental.pallas import tpu as pltpu
from jax.experimental.pallas import tpu_sc as plsc

BATCH = 16384
VOCABP1 = 100001
DIM = 64
FIELDS = 4
LANES = 16

_NC = 2
_NS = 16
_NW = _NC * _NS
_R = BATCH // _NW
_CG = DIM // LANES

_TBLK = 8192
_THALF = _TBLK // 2
_TGRID = (VOCABP1 + _TBLK - 1) // _TBLK

_mesh = plsc.VectorSubcoreMesh(core_axis_name="c", subcore_axis_name="s")
_params = pltpu.CompilerParams(use_tc_tiling_on_sc=False)


_LINROWS = _THALF * _TGRID


def _transpose_body(tt_ref, out_ref):
    x = tt_ref[...]
    x2 = jnp.concatenate([x[:, :_THALF], x[:, _THALF:]], axis=0)
    eye = jnp.eye(2 * DIM, dtype=jnp.float32)
    out_ref[...] = jax.lax.dot_general(x2, eye, (((0,), (0,)), ((), ())),
                                       preferred_element_type=jnp.float32)


_transpose_flat = pl.pallas_call(
    _transpose_body,
    grid=(_TGRID,),
    in_specs=[pl.BlockSpec((DIM, _TBLK), lambda j: (0, j))],
    out_specs=pl.BlockSpec((_THALF, 128), lambda j: (j, 0)),
    out_shape=jax.ShapeDtypeStruct((_LINROWS, 128), jnp.float32),
)


def _permute_indices(idx_v):
    def ibody(i, carry):
        v = idx_v[pl.ds(i * LANES, LANES)]
        t = v & (_TBLK - 1)
        two_t = t + t
        idx_v[pl.ds(i * LANES, LANES)] = (v - t) + jnp.where(
            t < _THALF, two_t, two_t - (_TBLK - 1))
        return carry

    lax.fori_loop(0, idx_v.shape[0] // LANES, ibody, 0)


_RC = 256
_NCH = _R // _RC


@functools.partial(
    pl.kernel,
    out_type=jax.ShapeDtypeStruct((BATCH, DIM), jnp.float32),
    mesh=_mesh,
    scratch_types=[
        [pltpu.VMEM((_RC,), jnp.int32) for _ in range(FIELDS)],
        [pltpu.VMEM((_RC, DIM), jnp.float32) for _ in range(FIELDS)],
        pltpu.SemaphoreType.DMA,
    ],
    compiler_params=_params,
)
def _embed_sum(l0, l1, l2, l3, t0, t1, t2, t3, out, idx_v, rows_v, sem):
    wid = lax.axis_index("s") * _NC + lax.axis_index("c")
    base = wid * _R
    labels = [l0, l1, l2, l3]
    tables = [t0, t1, t2, t3]

    for c in range(_NCH):
        row0 = base + c * _RC
        for f in range(FIELDS):
            pltpu.sync_copy(labels[f].at[pl.ds(row0, _RC)], idx_v[f])
            if tables[f].shape[0] != VOCABP1:
                _permute_indices(idx_v[f])
        descs = [
            pltpu.async_copy(tables[f].at[idx_v[f]], rows_v[f], sem)
            for f in range(FIELDS)
        ]
        for d in descs:
            d.wait()

        def body(r, carry):
            for cg in range(_CG):
                sl = pl.ds(cg * LANES, LANES)
                acc = (rows_v[0][r, sl] + rows_v[1][r, sl]
                       + rows_v[2][r, sl] + rows_v[3][r, sl])
                rows_v[0][r, sl] = acc
            return carry

        lax.fori_loop(0, _RC, body, 0)
        pltpu.sync_copy(rows_v[0], out.at[pl.ds(row0, _RC)])


def kernel(labels_f0, labels_f1, labels_f2, labels_f3,
           table_f0, table_f1, table_f2, table_f3):
    labels = [labels_f0, labels_f1, labels_f2, labels_f3]
    tables = [table_f0, table_f1, table_f2, table_f3]
    lins = [_transpose_flat(t.T).reshape(2 * _LINROWS, DIM) for t in tables]
    return _embed_sum(*labels, *lins)


# speedup vs baseline: 1.0685x; 1.0685x over previous
import functools

import jax
import jax.numpy as jnp
from jax import lax
from jax.experimental import pallas as pl
from jax.experimental.pallas import tpu as pltpu
from jax.experimental.pallas import tpu_sc as plsc

BATCH = 16384
VOCABP1 = 100001
DIM = 64
FIELDS = 4
LANES = 16

_NC = 2
_NS = 16
_NW = _NC * _NS
_R = BATCH // _NW
_CG = DIM // LANES

_TBLK = 16384
_THALF = _TBLK // 2
_TGRID = (VOCABP1 + _TBLK - 1) // _TBLK

_mesh = plsc.VectorSubcoreMesh(core_axis_name="c", subcore_axis_name="s")
_params = pltpu.CompilerParams(use_tc_tiling_on_sc=False)


_LINROWS = _THALF * _TGRID


def _transpose_body(tt_ref, out_ref):
    x = tt_ref[...]
    x2 = jnp.concatenate([x[:, :_THALF], x[:, _THALF:]], axis=0)
    eye = jnp.eye(2 * DIM, dtype=jnp.float32)
    out_ref[...] = jax.lax.dot_general(x2, eye, (((0,), (0,)), ((), ())),
                                       preferred_element_type=jnp.float32)


_transpose_flat = pl.pallas_call(
    _transpose_body,
    grid=(_TGRID,),
    in_specs=[pl.BlockSpec((DIM, _TBLK), lambda j: (0, j))],
    out_specs=pl.BlockSpec((_THALF, 128), lambda j: (j, 0)),
    out_shape=jax.ShapeDtypeStruct((_LINROWS, 128), jnp.float32),
)


def _permute_indices(idx_v):
    def ibody(i, carry):
        v = idx_v[pl.ds(i * LANES, LANES)]
        t = v & (_TBLK - 1)
        two_t = t + t
        idx_v[pl.ds(i * LANES, LANES)] = (v - t) + jnp.where(
            t < _THALF, two_t, two_t - (_TBLK - 1))
        return carry

    lax.fori_loop(0, idx_v.shape[0] // LANES, ibody, 0)


_RC = 256
_NCH = _R // _RC


@functools.partial(
    pl.kernel,
    out_type=jax.ShapeDtypeStruct((BATCH, DIM), jnp.float32),
    mesh=_mesh,
    scratch_types=[
        [pltpu.VMEM((_RC,), jnp.int32) for _ in range(FIELDS)],
        [pltpu.VMEM((_RC, DIM), jnp.float32) for _ in range(FIELDS)],
        pltpu.SemaphoreType.DMA,
    ],
    compiler_params=_params,
)
def _embed_sum(l0, l1, l2, l3, t0, t1, t2, t3, out, idx_v, rows_v, sem):
    wid = lax.axis_index("s") * _NC + lax.axis_index("c")
    base = wid * _R
    labels = [l0, l1, l2, l3]
    tables = [t0, t1, t2, t3]

    for c in range(_NCH):
        row0 = base + c * _RC
        for f in range(FIELDS):
            pltpu.sync_copy(labels[f].at[pl.ds(row0, _RC)], idx_v[f])
            if tables[f].shape[0] != VOCABP1:
                _permute_indices(idx_v[f])
        descs = [
            pltpu.async_copy(tables[f].at[idx_v[f]], rows_v[f], sem)
            for f in range(FIELDS)
        ]
        for d in descs:
            d.wait()

        def body(r, carry):
            for cg in range(_CG):
                sl = pl.ds(cg * LANES, LANES)
                acc = (rows_v[0][r, sl] + rows_v[1][r, sl]
                       + rows_v[2][r, sl] + rows_v[3][r, sl])
                rows_v[0][r, sl] = acc
            return carry

        lax.fori_loop(0, _RC, body, 0)
        pltpu.sync_copy(rows_v[0], out.at[pl.ds(row0, _RC)])


def kernel(labels_f0, labels_f1, labels_f2, labels_f3,
           table_f0, table_f1, table_f2, table_f3):
    labels = [labels_f0, labels_f1, labels_f2, labels_f3]
    tables = [table_f0, table_f1, table_f2, table_f3]
    lins = [_transpose_flat(t.T).reshape(2 * _LINROWS, DIM) for t in tables]
    return _embed_sum(*labels, *lins)
